# baseline (speedup 1.0000x reference)
import functools

import jax
import jax.numpy as jnp
from jax import lax
from jax.experimental import pallas as pl
from jax.experimental.pallas import tpu as pltpu
from jax.experimental.pallas import tpu_sc as plsc

B = 16384
NCTX = 28
PSTRIDE = 32
D = 256
E = 128
L = 16
NC, NS = 2, 16
NW = NC * NS
BPW = B // NW
C = 4
NCH = BPW // C


def _sc_pred_kernel(ctx_idx_hbm, winp_hbm, finp_pad_hbm, wr_hbm, w_hbm, wf_hbm,
                    out_hbm,
                    ctx_idx_v, winp_v, finp_v,
                    wr_rows0, wr_rows1, w_rows0, w_rows1, wf_rows0, wf_rows1,
                    pred_v,
                    sem_wr0, sem_wr1, sem_w0, sem_w1, sem_wf0, sem_wf1):
    wid = lax.axis_index("s") * NC + lax.axis_index("c")
    base = wid * BPW

    wr_rows = (wr_rows0, wr_rows1)
    w_rows = (w_rows0, w_rows1)
    wf_rows = (wf_rows0, wf_rows1)
    sem_wr = (sem_wr0, sem_wr1)
    sem_w = (sem_w0, sem_w1)
    sem_wf = (sem_wf0, sem_wf1)

    pltpu.sync_copy(ctx_idx_hbm.at[pl.ds(base * NCTX, BPW * NCTX)], ctx_idx_v)
    pltpu.sync_copy(winp_hbm.at[pl.ds(base * 6, BPW * 6)], winp_v)
    pltpu.sync_copy(finp_pad_hbm.at[pl.ds(base * 2, BPW * 2)], finp_v)

    third = jnp.float32(1.0 / 3.0)
    half = jnp.float32(0.5)
    lanes = lax.iota(jnp.int32, L)
    masks = [(lanes & bs) == 0 for bs in (1, 2, 4, 8)]
    perms = [lanes ^ bs for bs in (1, 2, 4, 8)]
    zero = jnp.zeros((L,), jnp.float32)

    def combine(a, b, stage):
        m, p = masks[stage], perms[stage]
        return jnp.where(m, a, b) + jnp.where(m, b, a)[p]

    def reduce_accs(accs):
        stage = 0
        while len(accs) > 1:
            accs = [combine(accs[i], accs[i + 1], stage)
                    for i in range(0, len(accs), 2)]
            stage += 1
        return accs[0]

    def gather_descs(ci, slot):
        return (
            pltpu.make_async_copy(
                wr_hbm.at[ctx_idx_v.at[pl.ds(ci * (C * NCTX), C * NCTX)]],
                wr_rows[slot], sem_wr[slot]),
            pltpu.make_async_copy(
                w_hbm.at[winp_v.at[pl.ds(ci * (C * 6), C * 6)]],
                w_rows[slot], sem_w[slot]),
            pltpu.make_async_copy(
                wf_hbm.at[finp_v.at[pl.ds(ci * (C * 2), C * 2)]],
                wf_rows[slot], sem_wf[slot]),
        )

    def issue(ci, slot):
        for d in gather_descs(ci, slot):
            d.start()

    def compute(ci, slot):
        for d in gather_descs(ci, slot):
            d.wait()

        @pl.loop(0, C, unroll=2)
        def _row(r, slot=slot, ci=ci):
            v = []
            for k in range(8):
                s = pl.ds(k * L, L)
                v.append((wf_rows[slot][2 * r, s]
                          + w_rows[slot][r * 6 + 0, s]
                          + w_rows[slot][r * 6 + 3, s]) * third)
            for k in range(8):
                s = pl.ds(k * L, L)
                s2 = pl.ds((k + 8) * L, L)
                acc = ((w_rows[slot][r * 6 + 1, s] + w_rows[slot][r * 6 + 2, s])
                       + (w_rows[slot][r * 6 + 4, s] + w_rows[slot][r * 6 + 5, s]))
                v.append((wf_rows[slot][2 * r, s2] + half * acc) * third)

            def dot_acc(j):
                row = r * NCTX + j
                part = [wr_rows[slot][row, pl.ds(k * L, L)] * v[k]
                        for k in range(16)]
                while len(part) > 1:
                    part = [part[i] + part[i + 1]
                            for i in range(0, len(part), 2)]
                return part[0]

            pbase = (ci * C + r) * PSTRIDE
            group_a = reduce_accs([dot_acc(j) for j in range(16)])
            pred_v[pl.ds(pbase, L)] = group_a
            group_b = reduce_accs([dot_acc(j) for j in range(16, NCTX)]
                                  + [zero] * (32 - NCTX))
            pred_v[pl.ds(pbase + L, L)] = group_b

    issue(0, 0)
    issue(1, 1)

    @pl.loop(0, NCH, step=2)
    def _chunk(ci):
        for b in range(2):
            cur = ci + b
            compute(cur, b)

            @pl.when(cur + 2 < NCH)
            def _issue_next(cur=cur, b=b):
                issue(cur + 2, b)

    pltpu.sync_copy(pred_v, out_hbm.at[pl.ds(base * PSTRIDE, BPW * PSTRIDE)])


@functools.partial(jax.jit, static_argnames=())
def _sc_pred(ctx_idx, winp, finp_pad, Wr, W, Wf):
    mesh = plsc.VectorSubcoreMesh(core_axis_name="c", subcore_axis_name="s",
                                  num_cores=NC, num_subcores=NS)
    f = pl.kernel(
        _sc_pred_kernel,
        out_type=jax.ShapeDtypeStruct((B * PSTRIDE,), jnp.float32),
        mesh=mesh,
        compiler_params=pltpu.CompilerParams(needs_layout_passes=False),
        scratch_types=[
            pltpu.VMEM((BPW * NCTX,), jnp.int32),
            pltpu.VMEM((BPW * 6,), jnp.int32),
            pltpu.VMEM((BPW * 2,), jnp.int32),
            pltpu.VMEM((C * NCTX, D), jnp.float32),
            pltpu.VMEM((C * NCTX, D), jnp.float32),
            pltpu.VMEM((C * 6, E), jnp.float32),
            pltpu.VMEM((C * 6, E), jnp.float32),
            pltpu.VMEM((C * 2, D), jnp.float32),
            pltpu.VMEM((C * 2, D), jnp.float32),
            pltpu.VMEM((BPW * PSTRIDE,), jnp.float32),
            pltpu.SemaphoreType.DMA,
            pltpu.SemaphoreType.DMA,
            pltpu.SemaphoreType.DMA,
            pltpu.SemaphoreType.DMA,
            pltpu.SemaphoreType.DMA,
            pltpu.SemaphoreType.DMA,
        ],
    )
    return f(ctx_idx, winp, finp_pad, Wr, W, Wf)


def _bce_kernel(pred_ref, out_ref):
    x = pred_ref[...]
    rows, cols = x.shape
    r = lax.broadcasted_iota(jnp.int32, (rows, cols), 0)
    c = lax.broadcasted_iota(jnp.int32, (rows, cols), 1)
    jpos = lax.rem(r * cols + c, PSTRIDE)
    label = (jpos < 3).astype(jnp.float32)
    valid = jpos < NCTX
    p = jax.nn.sigmoid(x)
    eps = 1e-12
    p = jnp.clip(p, eps, 1.0 - eps)
    term = -(label * jnp.log(p) + (1.0 - label) * jnp.log(1.0 - p))
    out_ref[0, 0] = jnp.sum(jnp.where(valid, term, 0.0))


def _tc_bce(pred2d):
    s = pl.pallas_call(
        _bce_kernel,
        out_shape=jax.ShapeDtypeStruct((1, 1), jnp.float32),
        out_specs=pl.BlockSpec(memory_space=pltpu.SMEM),
    )(pred2d)
    return s[0, 0] / jnp.float32(B * NCTX)


def kernel(inp, pos, neg, W, Wf, Wr):
    inp = inp.astype(jnp.int32)
    ctx_idx = jnp.concatenate([pos, neg], axis=1).astype(jnp.int32).reshape(-1)
    winp = inp[:, 1:7].reshape(-1)
    f0 = inp[:, 0]
    finp_pad = jnp.stack([f0, f0], axis=1).reshape(-1)
    pred = _sc_pred(ctx_idx, winp, finp_pad, Wr, W, Wf)
    return _tc_bce(pred.reshape(B * PSTRIDE // 128, 128))


# speedup vs baseline: 1.9140x; 1.9140x over previous
import functools

import jax
import jax.numpy as jnp
from jax import lax
from jax.experimental import pallas as pl
from jax.experimental.pallas import tpu as pltpu
from jax.experimental.pallas import tpu_sc as plsc

B = 16384
NCTX = 28
PSTRIDE = 32
D = 256
E = 128
L = 16
NC, NS = 2, 16
NW = NC * NS
BPW = B // NW
C = 4
NCH = BPW // C


def _sc_pred_kernel(ctx_idx_hbm, winp_hbm, finp_pad_hbm, wr_hbm, w_hbm, wf_hbm,
                    out_hbm,
                    ctx_idx_v, winp_v, finp_v,
                    wr_rows0, wr_rows1, w_rows0, w_rows1, wf_rows0, wf_rows1,
                    pred_v,
                    sem_wr0, sem_wr1, sem_w0, sem_w1, sem_wf0, sem_wf1):
    wid = lax.axis_index("s") * NC + lax.axis_index("c")
    base = wid * BPW

    wr_rows = (wr_rows0, wr_rows1)
    w_rows = (w_rows0, w_rows1)
    wf_rows = (wf_rows0, wf_rows1)
    sem_wr = (sem_wr0, sem_wr1)
    sem_w = (sem_w0, sem_w1)
    sem_wf = (sem_wf0, sem_wf1)

    pltpu.sync_copy(ctx_idx_hbm.at[pl.ds(base * NCTX, BPW * NCTX)], ctx_idx_v)
    pltpu.sync_copy(winp_hbm.at[pl.ds(base * 6, BPW * 6)], winp_v)
    pltpu.sync_copy(finp_pad_hbm.at[pl.ds(base * 2, BPW * 2)], finp_v)

    third = jnp.float32(1.0 / 3.0)
    half = jnp.float32(0.5)
    lanes = lax.iota(jnp.int32, L)
    masks = [(lanes & bs) == 0 for bs in (1, 2, 4, 8)]
    perms = [lanes ^ bs for bs in (1, 2, 4, 8)]
    zero = jnp.zeros((L,), jnp.float32)

    def combine(a, b, stage):
        m, p = masks[stage], perms[stage]
        return jnp.where(m, a, b) + jnp.where(m, b, a)[p]

    def reduce_accs(accs):
        stage = 0
        while len(accs) > 1:
            accs = [combine(accs[i], accs[i + 1], stage)
                    for i in range(0, len(accs), 2)]
            stage += 1
        return accs[0]

    def gather_descs(ci, slot):
        return (
            pltpu.make_async_copy(
                wr_hbm.at[ctx_idx_v.at[pl.ds(ci * (C * NCTX), C * NCTX)]],
                wr_rows[slot], sem_wr[slot]),
            pltpu.make_async_copy(
                w_hbm.at[winp_v.at[pl.ds(ci * (C * 6), C * 6)]],
                w_rows[slot], sem_w[slot]),
            pltpu.make_async_copy(
                wf_hbm.at[finp_v.at[pl.ds(ci * (C * 2), C * 2)]],
                wf_rows[slot], sem_wf[slot]),
        )

    def issue(ci, slot):
        for d in gather_descs(ci, slot):
            d.start()

    def compute(ci, slot):
        for d in gather_descs(ci, slot):
            d.wait()

        @pl.loop(0, C)
        def _row(r, slot=slot, ci=ci):
            v = []
            for k in range(8):
                s = pl.ds(k * L, L)
                v.append((wf_rows[slot][2 * r, s]
                          + w_rows[slot][r * 6 + 0, s]
                          + w_rows[slot][r * 6 + 3, s]) * third)
            for k in range(8):
                s = pl.ds(k * L, L)
                s2 = pl.ds((k + 8) * L, L)
                acc = ((w_rows[slot][r * 6 + 1, s] + w_rows[slot][r * 6 + 2, s])
                       + (w_rows[slot][r * 6 + 4, s] + w_rows[slot][r * 6 + 5, s]))
                v.append((wf_rows[slot][2 * r, s2] + half * acc) * third)

            def dot_acc(j):
                row = r * NCTX + j
                part = [wr_rows[slot][row, pl.ds(k * L, L)] * v[k]
                        for k in range(16)]
                while len(part) > 1:
                    part = [part[i] + part[i + 1]
                            for i in range(0, len(part), 2)]
                return part[0]

            pbase = (ci * C + r) * PSTRIDE
            group_a = reduce_accs([dot_acc(j) for j in range(16)])
            pred_v[pl.ds(pbase, L)] = group_a
            group_b = reduce_accs([dot_acc(j) for j in range(16, NCTX)]
                                  + [zero] * (32 - NCTX))
            pred_v[pl.ds(pbase + L, L)] = group_b

    issue(0, 0)
    issue(1, 1)

    @pl.loop(0, NCH, step=2)
    def _chunk(ci):
        for b in range(2):
            cur = ci + b
            compute(cur, b)

            @pl.when(cur + 2 < NCH)
            def _issue_next(cur=cur, b=b):
                issue(cur + 2, b)

    pltpu.sync_copy(pred_v, out_hbm.at[pl.ds(base * PSTRIDE, BPW * PSTRIDE)])


@functools.partial(jax.jit, static_argnames=())
def _sc_pred(ctx_idx, winp, finp_pad, Wr, W, Wf):
    mesh = plsc.VectorSubcoreMesh(core_axis_name="c", subcore_axis_name="s",
                                  num_cores=NC, num_subcores=NS)
    f = pl.kernel(
        _sc_pred_kernel,
        out_type=jax.ShapeDtypeStruct((B * PSTRIDE,), jnp.float32),
        mesh=mesh,
        compiler_params=pltpu.CompilerParams(needs_layout_passes=False),
        scratch_types=[
            pltpu.VMEM((BPW * NCTX,), jnp.int32),
            pltpu.VMEM((BPW * 6,), jnp.int32),
            pltpu.VMEM((BPW * 2,), jnp.int32),
            pltpu.VMEM((C * NCTX, D), jnp.float32),
            pltpu.VMEM((C * NCTX, D), jnp.float32),
            pltpu.VMEM((C * 6, E), jnp.float32),
            pltpu.VMEM((C * 6, E), jnp.float32),
            pltpu.VMEM((C * 2, D), jnp.float32),
            pltpu.VMEM((C * 2, D), jnp.float32),
            pltpu.VMEM((BPW * PSTRIDE,), jnp.float32),
            pltpu.SemaphoreType.DMA,
            pltpu.SemaphoreType.DMA,
            pltpu.SemaphoreType.DMA,
            pltpu.SemaphoreType.DMA,
            pltpu.SemaphoreType.DMA,
            pltpu.SemaphoreType.DMA,
        ],
    )
    return f(ctx_idx, winp, finp_pad, Wr, W, Wf)


def _bce_kernel(pred_ref, out_ref):
    x = pred_ref[...]
    rows, cols = x.shape
    r = lax.broadcasted_iota(jnp.int32, (rows, cols), 0)
    c = lax.broadcasted_iota(jnp.int32, (rows, cols), 1)
    jpos = lax.rem(r * cols + c, PSTRIDE)
    label = (jpos < 3).astype(jnp.float32)
    valid = jpos < NCTX
    p = jax.nn.sigmoid(x)
    eps = 1e-12
    p = jnp.clip(p, eps, 1.0 - eps)
    term = -(label * jnp.log(p) + (1.0 - label) * jnp.log(1.0 - p))
    out_ref[0, 0] = jnp.sum(jnp.where(valid, term, 0.0))


def _tc_bce(pred2d):
    s = pl.pallas_call(
        _bce_kernel,
        out_shape=jax.ShapeDtypeStruct((1, 1), jnp.float32),
        out_specs=pl.BlockSpec(memory_space=pltpu.SMEM),
    )(pred2d)
    return s[0, 0] / jnp.float32(B * NCTX)


def kernel(inp, pos, neg, W, Wf, Wr):
    inp = inp.astype(jnp.int32)
    ctx_idx = jnp.concatenate([pos, neg], axis=1).astype(jnp.int32).reshape(-1)
    winp = inp[:, 1:7].reshape(-1)
    f0 = inp[:, 0]
    finp_pad = jnp.stack([f0, f0], axis=1).reshape(-1)
    pred = _sc_pred(ctx_idx, winp, finp_pad, Wr, W, Wf)
    return _tc_bce(pred.reshape(B * PSTRIDE // 128, 128))
